# baseline (speedup 1.0000x reference)
import functools

import jax
import jax.numpy as jnp
from jax import lax
from jax.experimental import pallas as pl
from jax.experimental.pallas import tpu as pltpu
from jax.experimental.pallas import tpu_sc as plsc

B = 4096
S = 20
E = 128
HID = 1024
NCLS = 2

NC = 2
NS = 16
NW = NC * NS
CHUNK = 4
IDX_PER_CHUNK = CHUNK * S
L = 16

NSPLIT = 2
BSLICE = B // NSPLIT


def _gather_mean_body(b_per_w, idx_hbm, table_hbm, out_hbm,
                      idx_v, rows_v, out_v, sem):
    n_chunks = b_per_w // CHUNK
    wid = lax.axis_index("s") * NC + lax.axis_index("c")
    pltpu.sync_copy(idx_hbm.at[wid], idx_v)
    pltpu.async_copy(table_hbm.at[idx_v.at[0]], rows_v.at[0], sem)

    inv_s = jnp.float32(1.0 / S)

    def chunk_body(c, _):
        buf = lax.rem(c, 2)
        nxt = lax.rem(c + 1, 2)

        @pl.when(c + 1 < n_chunks)
        def _prefetch():
            pltpu.async_copy(table_hbm.at[idx_v.at[c + 1]], rows_v.at[nxt], sem)

        pltpu.make_async_copy(
            table_hbm.at[idx_v.at[c]], rows_v.at[buf], sem
        ).wait()

        for r in range(CHUNK):
            for g in range(E // L):
                sl = pl.ds(g * L, L)
                acc = rows_v[buf, r * S, sl]
                for j in range(1, S):
                    acc = acc + rows_v[buf, r * S + j, sl]
                out_v[c * CHUNK + r, sl] = acc * inv_s
        return 0

    lax.fori_loop(0, n_chunks, chunk_body, 0)
    pltpu.sync_copy(out_v, out_hbm.at[pl.ds(wid * b_per_w, b_per_w)])


@functools.cache
def _gather_mean(bslice):
    b_per_w = bslice // NW
    n_chunks = b_per_w // CHUNK
    mesh = plsc.VectorSubcoreMesh(core_axis_name="c", subcore_axis_name="s")
    return pl.kernel(
        functools.partial(_gather_mean_body, b_per_w),
        mesh=mesh,
        out_type=jax.ShapeDtypeStruct((bslice, E), jnp.float32),
        scratch_types=[
            pltpu.VMEM((n_chunks, IDX_PER_CHUNK), jnp.int32),
            pltpu.VMEM((2, IDX_PER_CHUNK, E), jnp.float32),
            pltpu.VMEM((b_per_w, E), jnp.float32),
            pltpu.SemaphoreType.DMA,
        ],
    )


def _mlp_body(x_ref, w1_ref, b1_ref, w2_ref, b2_ref, w3_ref, b3_ref, o_ref):
    dn = (((1,), (1,)), ((), ()))
    x = x_ref[...]
    h = lax.dot_general(x, w1_ref[...], dn, preferred_element_type=jnp.float32)
    h = jnp.maximum(h + b1_ref[...], 0.0)
    h = lax.dot_general(h, w2_ref[...], dn, preferred_element_type=jnp.float32)
    h = jnp.maximum(h + b2_ref[...], 0.0)
    logits = lax.dot_general(h, w3_ref[...], dn,
                             preferred_element_type=jnp.float32)
    logits = logits + b3_ref[...]
    m = jnp.max(logits, axis=-1, keepdims=True)
    sh = logits - m
    lse = jnp.log(jnp.sum(jnp.exp(sh), axis=-1, keepdims=True))
    o_ref[...] = sh - lse


BB = 512


def _mlp(avg, W1, b1, W2, b2, W3, b3):
    bsz = avg.shape[0]
    grid = (bsz // BB,)
    return pl.pallas_call(
        _mlp_body,
        grid=grid,
        in_specs=[
            pl.BlockSpec((BB, E), lambda i: (i, 0)),
            pl.BlockSpec((HID, E), lambda i: (0, 0)),
            pl.BlockSpec((1, HID), lambda i: (0, 0)),
            pl.BlockSpec((HID, HID), lambda i: (0, 0)),
            pl.BlockSpec((1, HID), lambda i: (0, 0)),
            pl.BlockSpec((NCLS, HID), lambda i: (0, 0)),
            pl.BlockSpec((1, NCLS), lambda i: (0, 0)),
        ],
        out_specs=pl.BlockSpec((BB, NCLS), lambda i: (i, 0)),
        out_shape=jax.ShapeDtypeStruct((bsz, NCLS), jnp.float32),
    )(avg, W1, b1, W2, b2, W3, b3)


def kernel(word_indices, emb_table, W1, b1, W2, b2, W3, b3):
    idx = word_indices.reshape(NSPLIT, NW, BSLICE // NW // CHUNK,
                               IDX_PER_CHUNK).astype(jnp.int32)
    b1r = b1.reshape(1, HID)
    b2r = b2.reshape(1, HID)
    b3r = b3.reshape(1, NCLS)
    outs = []
    for p in range(NSPLIT):
        avg = _gather_mean(BSLICE)(idx[p], emb_table)
        outs.append(_mlp(avg, W1, b1r, W2, b2r, W3, b3r))
    return jnp.concatenate(outs, axis=0)


# speedup vs baseline: 1.0488x; 1.0488x over previous
import functools

import jax
import jax.numpy as jnp
from jax import lax
from jax.experimental import pallas as pl
from jax.experimental.pallas import tpu as pltpu
from jax.experimental.pallas import tpu_sc as plsc

B = 4096
S = 20
E = 128
HID = 1024
NCLS = 2

NC = 2
NS = 16
NW = NC * NS
CHUNK = 4
IDX_PER_CHUNK = CHUNK * S
L = 16
NBUF = 4

NSPLIT = 2
BSLICE = B // NSPLIT


def _gather_mean_body(b_per_w, idx_hbm, table_hbm, out_hbm,
                      idx_v, rows_v, out_v, sem):
    n_chunks = b_per_w // CHUNK
    wid = lax.axis_index("s") * NC + lax.axis_index("c")
    pltpu.sync_copy(idx_hbm.at[wid], idx_v)
    for p in range(NBUF - 1):
        pltpu.async_copy(table_hbm.at[idx_v.at[p]], rows_v.at[p], sem)

    inv_s = jnp.float32(1.0 / S)

    def chunk_body(c, _):
        buf = lax.rem(c, NBUF)
        nxt = lax.rem(c + NBUF - 1, NBUF)

        @pl.when(c + NBUF - 1 < n_chunks)
        def _prefetch():
            pltpu.async_copy(table_hbm.at[idx_v.at[c + NBUF - 1]],
                             rows_v.at[nxt], sem)

        pltpu.make_async_copy(
            table_hbm.at[idx_v.at[c]], rows_v.at[buf], sem
        ).wait()

        for r in range(CHUNK):
            for g in range(E // L):
                sl = pl.ds(g * L, L)
                acc = rows_v[buf, r * S, sl]
                for j in range(1, S):
                    acc = acc + rows_v[buf, r * S + j, sl]
                out_v[c * CHUNK + r, sl] = acc * inv_s
        return 0

    lax.fori_loop(0, n_chunks, chunk_body, 0)
    pltpu.sync_copy(out_v, out_hbm.at[pl.ds(wid * b_per_w, b_per_w)])


@functools.cache
def _gather_mean(bslice):
    b_per_w = bslice // NW
    n_chunks = b_per_w // CHUNK
    mesh = plsc.VectorSubcoreMesh(core_axis_name="c", subcore_axis_name="s")
    return pl.kernel(
        functools.partial(_gather_mean_body, b_per_w),
        mesh=mesh,
        out_type=jax.ShapeDtypeStruct((bslice, E), jnp.float32),
        scratch_types=[
            pltpu.VMEM((n_chunks, IDX_PER_CHUNK), jnp.int32),
            pltpu.VMEM((NBUF, IDX_PER_CHUNK, E), jnp.float32),
            pltpu.VMEM((b_per_w, E), jnp.float32),
            pltpu.SemaphoreType.DMA,
        ],
    )


def _mlp_body(x_ref, w1_ref, b1_ref, w2_ref, b2_ref, w3_ref, b3_ref, o_ref):
    dn = (((1,), (1,)), ((), ()))
    x = x_ref[...]
    h = lax.dot_general(x, w1_ref[...], dn, preferred_element_type=jnp.float32)
    h = jnp.maximum(h + b1_ref[...], 0.0)
    h = lax.dot_general(h, w2_ref[...], dn, preferred_element_type=jnp.float32)
    h = jnp.maximum(h + b2_ref[...], 0.0)
    logits = lax.dot_general(h, w3_ref[...], dn,
                             preferred_element_type=jnp.float32)
    logits = logits + b3_ref[...]
    m = jnp.max(logits, axis=-1, keepdims=True)
    sh = logits - m
    lse = jnp.log(jnp.sum(jnp.exp(sh), axis=-1, keepdims=True))
    o_ref[...] = sh - lse


BB = 512


def _mlp(avg, W1, b1, W2, b2, W3, b3):
    bsz = avg.shape[0]
    grid = (bsz // BB,)
    return pl.pallas_call(
        _mlp_body,
        grid=grid,
        in_specs=[
            pl.BlockSpec((BB, E), lambda i: (i, 0)),
            pl.BlockSpec((HID, E), lambda i: (0, 0)),
            pl.BlockSpec((1, HID), lambda i: (0, 0)),
            pl.BlockSpec((HID, HID), lambda i: (0, 0)),
            pl.BlockSpec((1, HID), lambda i: (0, 0)),
            pl.BlockSpec((NCLS, HID), lambda i: (0, 0)),
            pl.BlockSpec((1, NCLS), lambda i: (0, 0)),
        ],
        out_specs=pl.BlockSpec((BB, NCLS), lambda i: (i, 0)),
        out_shape=jax.ShapeDtypeStruct((bsz, NCLS), jnp.float32),
    )(avg, W1, b1, W2, b2, W3, b3)


def kernel(word_indices, emb_table, W1, b1, W2, b2, W3, b3):
    idx = word_indices.reshape(NSPLIT, NW, BSLICE // NW // CHUNK,
                               IDX_PER_CHUNK).astype(jnp.int32)
    b1r = b1.reshape(1, HID)
    b2r = b2.reshape(1, HID)
    b3r = b3.reshape(1, NCLS)
    outs = []
    for p in range(NSPLIT):
        avg = _gather_mean(BSLICE)(idx[p], emb_table)
        outs.append(_mlp(avg, W1, b1r, W2, b2r, W3, b3r))
    return jnp.concatenate(outs, axis=0)


# speedup vs baseline: 1.2042x; 1.1482x over previous
import functools

import jax
import jax.numpy as jnp
from jax import lax
from jax.experimental import pallas as pl
from jax.experimental.pallas import tpu as pltpu
from jax.experimental.pallas import tpu_sc as plsc

B = 4096
S = 20
E = 128
HID = 1024
NCLS = 2

NC = 2
NS = 16
NW = NC * NS
CHUNK = 4
IDX_PER_CHUNK = CHUNK * S
L = 16
NBUF = 4

NSPLIT = 2
BSLICE = B // NSPLIT


def _gather_mean_body(b_per_w, idx_hbm, table_hbm, out_hbm,
                      idx_v, rows_v, out_v, sem):
    n_chunks = b_per_w // CHUNK
    wid = lax.axis_index("s") * NC + lax.axis_index("c")
    pltpu.sync_copy(idx_hbm.at[wid], idx_v)
    for p in range(NBUF - 1):
        pltpu.async_copy(table_hbm.at[idx_v.at[p]], rows_v.at[p], sem)

    inv_s = jnp.float32(1.0 / S)

    def chunk_body(c, _):
        buf = lax.rem(c, NBUF)
        nxt = lax.rem(c + NBUF - 1, NBUF)

        @pl.when(c + NBUF - 1 < n_chunks)
        def _prefetch():
            pltpu.async_copy(table_hbm.at[idx_v.at[c + NBUF - 1]],
                             rows_v.at[nxt], sem)

        pltpu.make_async_copy(
            table_hbm.at[idx_v.at[c]], rows_v.at[buf], sem
        ).wait()

        for r in range(CHUNK):
            base = r * S
            for g in range(E // L):
                sl = pl.ds(g * L, L)
                a0 = rows_v[buf, base + 0, sl]
                a1 = rows_v[buf, base + 1, sl]
                a2 = rows_v[buf, base + 2, sl]
                a3 = rows_v[buf, base + 3, sl]
                for j in range(4, S, 4):
                    a0 = a0 + rows_v[buf, base + j + 0, sl]
                    a1 = a1 + rows_v[buf, base + j + 1, sl]
                    a2 = a2 + rows_v[buf, base + j + 2, sl]
                    a3 = a3 + rows_v[buf, base + j + 3, sl]
                out_v[c * CHUNK + r, sl] = ((a0 + a1) + (a2 + a3)) * inv_s
        return 0

    lax.fori_loop(0, n_chunks, chunk_body, 0)
    pltpu.sync_copy(out_v, out_hbm.at[pl.ds(wid * b_per_w, b_per_w)])


@functools.cache
def _gather_mean(bslice):
    b_per_w = bslice // NW
    n_chunks = b_per_w // CHUNK
    mesh = plsc.VectorSubcoreMesh(core_axis_name="c", subcore_axis_name="s")
    return pl.kernel(
        functools.partial(_gather_mean_body, b_per_w),
        mesh=mesh,
        out_type=jax.ShapeDtypeStruct((bslice, E), jnp.float32),
        scratch_types=[
            pltpu.VMEM((n_chunks, IDX_PER_CHUNK), jnp.int32),
            pltpu.VMEM((NBUF, IDX_PER_CHUNK, E), jnp.float32),
            pltpu.VMEM((b_per_w, E), jnp.float32),
            pltpu.SemaphoreType.DMA,
        ],
    )


def _mlp_body(x_ref, w1_ref, b1_ref, w2_ref, b2_ref, w3_ref, b3_ref, o_ref):
    dn = (((1,), (1,)), ((), ()))
    x = x_ref[...]
    h = lax.dot_general(x, w1_ref[...], dn, preferred_element_type=jnp.float32)
    h = jnp.maximum(h + b1_ref[...], 0.0)
    h = lax.dot_general(h, w2_ref[...], dn, preferred_element_type=jnp.float32)
    h = jnp.maximum(h + b2_ref[...], 0.0)
    logits = lax.dot_general(h, w3_ref[...], dn,
                             preferred_element_type=jnp.float32)
    logits = logits + b3_ref[...]
    m = jnp.max(logits, axis=-1, keepdims=True)
    sh = logits - m
    lse = jnp.log(jnp.sum(jnp.exp(sh), axis=-1, keepdims=True))
    o_ref[...] = sh - lse


BB = 512


def _mlp(avg, W1, b1, W2, b2, W3, b3):
    bsz = avg.shape[0]
    grid = (bsz // BB,)
    return pl.pallas_call(
        _mlp_body,
        grid=grid,
        in_specs=[
            pl.BlockSpec((BB, E), lambda i: (i, 0)),
            pl.BlockSpec((HID, E), lambda i: (0, 0)),
            pl.BlockSpec((1, HID), lambda i: (0, 0)),
            pl.BlockSpec((HID, HID), lambda i: (0, 0)),
            pl.BlockSpec((1, HID), lambda i: (0, 0)),
            pl.BlockSpec((NCLS, HID), lambda i: (0, 0)),
            pl.BlockSpec((1, NCLS), lambda i: (0, 0)),
        ],
        out_specs=pl.BlockSpec((BB, NCLS), lambda i: (i, 0)),
        out_shape=jax.ShapeDtypeStruct((bsz, NCLS), jnp.float32),
    )(avg, W1, b1, W2, b2, W3, b3)


def kernel(word_indices, emb_table, W1, b1, W2, b2, W3, b3):
    idx = word_indices.reshape(NSPLIT, NW, BSLICE // NW // CHUNK,
                               IDX_PER_CHUNK).astype(jnp.int32)
    b1r = b1.reshape(1, HID)
    b2r = b2.reshape(1, HID)
    b3r = b3.reshape(1, NCLS)
    outs = []
    for p in range(NSPLIT):
        avg = _gather_mean(BSLICE)(idx[p], emb_table)
        outs.append(_mlp(avg, W1, b1r, W2, b2r, W3, b3r))
    return jnp.concatenate(outs, axis=0)


# speedup vs baseline: 1.2254x; 1.0175x over previous
import functools

import jax
import jax.numpy as jnp
from jax import lax
from jax.experimental import pallas as pl
from jax.experimental.pallas import tpu as pltpu
from jax.experimental.pallas import tpu_sc as plsc

B = 4096
S = 20
E = 128
HID = 1024
NCLS = 2

NC = 2
NS = 16
NW = NC * NS
CHUNK = 4
IDX_PER_CHUNK = CHUNK * S
L = 16
NBUF = 4

NSPLIT = 2
BSLICE = B // NSPLIT


def _gather_mean_body(b_per_w, idx_hbm, table_hbm, out_hbm,
                      idx_v, rows_v, out_v, sem):
    n_chunks = b_per_w // CHUNK
    wid = lax.axis_index("s") * NC + lax.axis_index("c")
    pltpu.sync_copy(idx_hbm.at[wid], idx_v)
    for p in range(NBUF - 1):
        pltpu.async_copy(table_hbm.at[idx_v.at[p]], rows_v.at[p], sem)

    inv_s = jnp.float32(1.0 / S)

    def chunk_body(c, _):
        buf = lax.rem(c, NBUF)
        nxt = lax.rem(c + NBUF - 1, NBUF)

        @pl.when(c + NBUF - 1 < n_chunks)
        def _prefetch():
            pltpu.async_copy(table_hbm.at[idx_v.at[c + NBUF - 1]],
                             rows_v.at[nxt], sem)

        pltpu.make_async_copy(
            table_hbm.at[idx_v.at[c]], rows_v.at[buf], sem
        ).wait()

        for r in range(CHUNK):
            base = r * S
            for g in range(E // L):
                sl = pl.ds(g * L, L)
                a0 = rows_v[buf, base + 0, sl]
                a1 = rows_v[buf, base + 1, sl]
                a2 = rows_v[buf, base + 2, sl]
                a3 = rows_v[buf, base + 3, sl]
                for j in range(4, S, 4):
                    a0 = a0 + rows_v[buf, base + j + 0, sl]
                    a1 = a1 + rows_v[buf, base + j + 1, sl]
                    a2 = a2 + rows_v[buf, base + j + 2, sl]
                    a3 = a3 + rows_v[buf, base + j + 3, sl]
                out_v[c * CHUNK + r, sl] = ((a0 + a1) + (a2 + a3)) * inv_s
        return 0

    lax.fori_loop(0, n_chunks, chunk_body, 0)
    pltpu.sync_copy(out_v, out_hbm.at[pl.ds(wid * b_per_w, b_per_w)])


@functools.cache
def _gather_mean(bslice):
    b_per_w = bslice // NW
    n_chunks = b_per_w // CHUNK
    mesh = plsc.VectorSubcoreMesh(core_axis_name="c", subcore_axis_name="s")
    return pl.kernel(
        functools.partial(_gather_mean_body, b_per_w),
        mesh=mesh,
        out_type=jax.ShapeDtypeStruct((bslice, E), jnp.float32),
        scratch_types=[
            pltpu.VMEM((n_chunks, IDX_PER_CHUNK), jnp.int32),
            pltpu.VMEM((NBUF, IDX_PER_CHUNK, E), jnp.float32),
            pltpu.VMEM((b_per_w, E), jnp.float32),
            pltpu.SemaphoreType.DMA,
        ],
    )


def _mlp_body(x_ref, w1_ref, b1_ref, w2_ref, b2_ref, w3_ref, b3_ref, o_ref):
    dn = (((1,), (1,)), ((), ()))
    x = x_ref[...]
    h = lax.dot_general(x, w1_ref[...], dn, preferred_element_type=jnp.float32)
    h = jnp.maximum(h + b1_ref[...], 0.0)
    h = lax.dot_general(h, w2_ref[...], dn, preferred_element_type=jnp.float32)
    h = jnp.maximum(h + b2_ref[...], 0.0)
    logits = lax.dot_general(h, w3_ref[...], dn,
                             preferred_element_type=jnp.float32)
    logits = logits + b3_ref[...]
    m = jnp.max(logits, axis=-1, keepdims=True)
    sh = logits - m
    lse = jnp.log(jnp.sum(jnp.exp(sh), axis=-1, keepdims=True))
    o_ref[...] = sh - lse


BB = 1024


def _mlp(avg, W1, b1, W2, b2, W3, b3):
    bsz = avg.shape[0]
    grid = (bsz // BB,)
    return pl.pallas_call(
        _mlp_body,
        grid=grid,
        in_specs=[
            pl.BlockSpec((BB, E), lambda i: (i, 0)),
            pl.BlockSpec((HID, E), lambda i: (0, 0)),
            pl.BlockSpec((1, HID), lambda i: (0, 0)),
            pl.BlockSpec((HID, HID), lambda i: (0, 0)),
            pl.BlockSpec((1, HID), lambda i: (0, 0)),
            pl.BlockSpec((NCLS, HID), lambda i: (0, 0)),
            pl.BlockSpec((1, NCLS), lambda i: (0, 0)),
        ],
        out_specs=pl.BlockSpec((BB, NCLS), lambda i: (i, 0)),
        out_shape=jax.ShapeDtypeStruct((bsz, NCLS), jnp.float32),
    )(avg, W1, b1, W2, b2, W3, b3)


def kernel(word_indices, emb_table, W1, b1, W2, b2, W3, b3):
    idx = word_indices.reshape(NSPLIT, NW, BSLICE // NW // CHUNK,
                               IDX_PER_CHUNK).astype(jnp.int32)
    b1r = b1.reshape(1, HID)
    b2r = b2.reshape(1, HID)
    b3r = b3.reshape(1, NCLS)
    outs = []
    for p in range(NSPLIT):
        avg = _gather_mean(BSLICE)(idx[p], emb_table)
        outs.append(_mlp(avg, W1, b1r, W2, b2r, W3, b3r))
    return jnp.concatenate(outs, axis=0)


# speedup vs baseline: 1.2318x; 1.0053x over previous
import functools

import jax
import jax.numpy as jnp
from jax import lax
from jax.experimental import pallas as pl
from jax.experimental.pallas import tpu as pltpu
from jax.experimental.pallas import tpu_sc as plsc

B = 4096
S = 20
E = 128
HID = 1024
NCLS = 2

NC = 2
NS = 16
NW = NC * NS
CHUNK = 4
IDX_PER_CHUNK = CHUNK * S
L = 16
NBUF = 6

NSPLIT = 2
BSLICE = B // NSPLIT


def _gather_mean_body(b_per_w, idx_hbm, table_hbm, out_hbm,
                      idx_v, rows_v, out_v, sem):
    n_chunks = b_per_w // CHUNK
    wid = lax.axis_index("s") * NC + lax.axis_index("c")
    pltpu.sync_copy(idx_hbm.at[wid], idx_v)
    for p in range(NBUF - 1):
        pltpu.async_copy(table_hbm.at[idx_v.at[p]], rows_v.at[p], sem)

    inv_s = jnp.float32(1.0 / S)

    def chunk_body(c, _):
        buf = lax.rem(c, NBUF)
        nxt = lax.rem(c + NBUF - 1, NBUF)

        @pl.when(c + NBUF - 1 < n_chunks)
        def _prefetch():
            pltpu.async_copy(table_hbm.at[idx_v.at[c + NBUF - 1]],
                             rows_v.at[nxt], sem)

        pltpu.make_async_copy(
            table_hbm.at[idx_v.at[c]], rows_v.at[buf], sem
        ).wait()

        for r in range(CHUNK):
            base = r * S
            for g in range(E // L):
                sl = pl.ds(g * L, L)
                a0 = rows_v[buf, base + 0, sl]
                a1 = rows_v[buf, base + 1, sl]
                a2 = rows_v[buf, base + 2, sl]
                a3 = rows_v[buf, base + 3, sl]
                for j in range(4, S, 4):
                    a0 = a0 + rows_v[buf, base + j + 0, sl]
                    a1 = a1 + rows_v[buf, base + j + 1, sl]
                    a2 = a2 + rows_v[buf, base + j + 2, sl]
                    a3 = a3 + rows_v[buf, base + j + 3, sl]
                out_v[c * CHUNK + r, sl] = ((a0 + a1) + (a2 + a3)) * inv_s
        return 0

    lax.fori_loop(0, n_chunks, chunk_body, 0)
    pltpu.sync_copy(out_v, out_hbm.at[pl.ds(wid * b_per_w, b_per_w)])


@functools.cache
def _gather_mean(bslice):
    b_per_w = bslice // NW
    n_chunks = b_per_w // CHUNK
    mesh = plsc.VectorSubcoreMesh(core_axis_name="c", subcore_axis_name="s")
    return pl.kernel(
        functools.partial(_gather_mean_body, b_per_w),
        mesh=mesh,
        out_type=jax.ShapeDtypeStruct((bslice, E), jnp.float32),
        scratch_types=[
            pltpu.VMEM((n_chunks, IDX_PER_CHUNK), jnp.int32),
            pltpu.VMEM((NBUF, IDX_PER_CHUNK, E), jnp.float32),
            pltpu.VMEM((b_per_w, E), jnp.float32),
            pltpu.SemaphoreType.DMA,
        ],
    )


def _mlp_body(x_ref, w1_ref, b1_ref, w2_ref, b2_ref, w3_ref, b3_ref, o_ref):
    dn = (((1,), (1,)), ((), ()))
    x = x_ref[...]
    h = lax.dot_general(x, w1_ref[...], dn, preferred_element_type=jnp.float32)
    h = jnp.maximum(h + b1_ref[...], 0.0)
    h = lax.dot_general(h, w2_ref[...], dn, preferred_element_type=jnp.float32)
    h = jnp.maximum(h + b2_ref[...], 0.0)
    logits = lax.dot_general(h, w3_ref[...], dn,
                             preferred_element_type=jnp.float32)
    logits = logits + b3_ref[...]
    m = jnp.max(logits, axis=-1, keepdims=True)
    sh = logits - m
    lse = jnp.log(jnp.sum(jnp.exp(sh), axis=-1, keepdims=True))
    o_ref[...] = sh - lse


BB = 1024


def _mlp(avg, W1, b1, W2, b2, W3, b3):
    bsz = avg.shape[0]
    grid = (bsz // BB,)
    return pl.pallas_call(
        _mlp_body,
        grid=grid,
        in_specs=[
            pl.BlockSpec((BB, E), lambda i: (i, 0)),
            pl.BlockSpec((HID, E), lambda i: (0, 0)),
            pl.BlockSpec((1, HID), lambda i: (0, 0)),
            pl.BlockSpec((HID, HID), lambda i: (0, 0)),
            pl.BlockSpec((1, HID), lambda i: (0, 0)),
            pl.BlockSpec((NCLS, HID), lambda i: (0, 0)),
            pl.BlockSpec((1, NCLS), lambda i: (0, 0)),
        ],
        out_specs=pl.BlockSpec((BB, NCLS), lambda i: (i, 0)),
        out_shape=jax.ShapeDtypeStruct((bsz, NCLS), jnp.float32),
    )(avg, W1, b1, W2, b2, W3, b3)


def kernel(word_indices, emb_table, W1, b1, W2, b2, W3, b3):
    idx = word_indices.reshape(NSPLIT, NW, BSLICE // NW // CHUNK,
                               IDX_PER_CHUNK).astype(jnp.int32)
    b1r = b1.reshape(1, HID)
    b2r = b2.reshape(1, HID)
    b3r = b3.reshape(1, NCLS)
    outs = []
    for p in range(NSPLIT):
        avg = _gather_mean(BSLICE)(idx[p], emb_table)
        outs.append(_mlp(avg, W1, b1r, W2, b2r, W3, b3r))
    return jnp.concatenate(outs, axis=0)


# speedup vs baseline: 1.2347x; 1.0023x over previous
import functools

import jax
import jax.numpy as jnp
from jax import lax
from jax.experimental import pallas as pl
from jax.experimental.pallas import tpu as pltpu
from jax.experimental.pallas import tpu_sc as plsc

B = 4096
S = 20
E = 128
HID = 1024
NCLS = 2

NC = 2
NS = 16
NW = NC * NS
CHUNK = 4
IDX_PER_CHUNK = CHUNK * S
L = 16
NBUF = 6

SPLITS = (3072, 1024)


def _gather_mean_body(b_per_w, idx_hbm, table_hbm, out_hbm,
                      idx_v, rows_v, out_v, sem):
    n_chunks = b_per_w // CHUNK
    wid = lax.axis_index("s") * NC + lax.axis_index("c")
    pltpu.sync_copy(idx_hbm.at[wid], idx_v)
    for p in range(NBUF - 1):
        pltpu.async_copy(table_hbm.at[idx_v.at[p]], rows_v.at[p], sem)

    inv_s = jnp.float32(1.0 / S)

    def chunk_body(c, _):
        buf = lax.rem(c, NBUF)
        nxt = lax.rem(c + NBUF - 1, NBUF)

        @pl.when(c + NBUF - 1 < n_chunks)
        def _prefetch():
            pltpu.async_copy(table_hbm.at[idx_v.at[c + NBUF - 1]],
                             rows_v.at[nxt], sem)

        pltpu.make_async_copy(
            table_hbm.at[idx_v.at[c]], rows_v.at[buf], sem
        ).wait()

        for r in range(CHUNK):
            base = r * S
            for g in range(E // L):
                sl = pl.ds(g * L, L)
                a0 = rows_v[buf, base + 0, sl]
                a1 = rows_v[buf, base + 1, sl]
                a2 = rows_v[buf, base + 2, sl]
                a3 = rows_v[buf, base + 3, sl]
                for j in range(4, S, 4):
                    a0 = a0 + rows_v[buf, base + j + 0, sl]
                    a1 = a1 + rows_v[buf, base + j + 1, sl]
                    a2 = a2 + rows_v[buf, base + j + 2, sl]
                    a3 = a3 + rows_v[buf, base + j + 3, sl]
                out_v[c * CHUNK + r, sl] = ((a0 + a1) + (a2 + a3)) * inv_s
        return 0

    lax.fori_loop(0, n_chunks, chunk_body, 0)
    pltpu.sync_copy(out_v, out_hbm.at[pl.ds(wid * b_per_w, b_per_w)])


@functools.cache
def _gather_mean(bslice):
    b_per_w = bslice // NW
    n_chunks = b_per_w // CHUNK
    mesh = plsc.VectorSubcoreMesh(core_axis_name="c", subcore_axis_name="s")
    return pl.kernel(
        functools.partial(_gather_mean_body, b_per_w),
        mesh=mesh,
        out_type=jax.ShapeDtypeStruct((bslice, E), jnp.float32),
        scratch_types=[
            pltpu.VMEM((n_chunks, IDX_PER_CHUNK), jnp.int32),
            pltpu.VMEM((NBUF, IDX_PER_CHUNK, E), jnp.float32),
            pltpu.VMEM((b_per_w, E), jnp.float32),
            pltpu.SemaphoreType.DMA,
        ],
    )


def _mlp_body(x_ref, w1_ref, b1_ref, w2_ref, b2_ref, w3_ref, b3_ref, o_ref):
    dn = (((1,), (1,)), ((), ()))
    x = x_ref[...]
    h = lax.dot_general(x, w1_ref[...], dn, preferred_element_type=jnp.float32)
    h = jnp.maximum(h + b1_ref[...], 0.0)
    h = lax.dot_general(h, w2_ref[...], dn, preferred_element_type=jnp.float32)
    h = jnp.maximum(h + b2_ref[...], 0.0)
    logits = lax.dot_general(h, w3_ref[...], dn,
                             preferred_element_type=jnp.float32)
    logits = logits + b3_ref[...]
    m = jnp.max(logits, axis=-1, keepdims=True)
    sh = logits - m
    lse = jnp.log(jnp.sum(jnp.exp(sh), axis=-1, keepdims=True))
    o_ref[...] = sh - lse


BB = 1024


def _mlp(avg, W1, b1, W2, b2, W3, b3):
    bsz = avg.shape[0]
    grid = (bsz // BB,)
    return pl.pallas_call(
        _mlp_body,
        grid=grid,
        in_specs=[
            pl.BlockSpec((BB, E), lambda i: (i, 0)),
            pl.BlockSpec((HID, E), lambda i: (0, 0)),
            pl.BlockSpec((1, HID), lambda i: (0, 0)),
            pl.BlockSpec((HID, HID), lambda i: (0, 0)),
            pl.BlockSpec((1, HID), lambda i: (0, 0)),
            pl.BlockSpec((NCLS, HID), lambda i: (0, 0)),
            pl.BlockSpec((1, NCLS), lambda i: (0, 0)),
        ],
        out_specs=pl.BlockSpec((BB, NCLS), lambda i: (i, 0)),
        out_shape=jax.ShapeDtypeStruct((bsz, NCLS), jnp.float32),
    )(avg, W1, b1, W2, b2, W3, b3)


def kernel(word_indices, emb_table, W1, b1, W2, b2, W3, b3):
    idx_flat = word_indices.reshape(-1).astype(jnp.int32)
    b1r = b1.reshape(1, HID)
    b2r = b2.reshape(1, HID)
    b3r = b3.reshape(1, NCLS)
    outs = []
    off = 0
    for bslice in SPLITS:
        n_idx = bslice * S
        idx_p = lax.dynamic_slice(idx_flat, (off,), (n_idx,)).reshape(
            NW, bslice // NW // CHUNK, IDX_PER_CHUNK)
        avg = _gather_mean(bslice)(idx_p, emb_table)
        outs.append(_mlp(avg, W1, b1r, W2, b2r, W3, b3r))
        off += n_idx
    return jnp.concatenate(outs, axis=0)
